# Initial kernel scaffold; baseline (speedup 1.0000x reference)
import jax
import jax.numpy as jnp
from jax.experimental import pallas as pl


def kernel(x):
    raise NotImplementedError("write your pallas kernel here")


# speedup vs baseline: 3.4849x; 3.4849x over previous
import jax
import jax.numpy as jnp
from jax.experimental import pallas as pl

TOPK = 64
ROWS_PER_BLOCK = 8
SEARCH_BITS = 31


def _abs_topk_block(x_ref, o_ref):
    x = x_ref[...]
    b = jax.lax.bitcast_convert_type(x, jnp.int32) & jnp.int32(0x7FFFFFFF)
    r = x.shape[0]

    def body(_, lohi):
        lo, hi = lohi
        mid = lo + ((hi - lo) >> 1)
        cnt = jnp.sum((b >= mid).astype(jnp.int32), axis=1, keepdims=True)
        ge = cnt >= TOPK
        return jnp.where(ge, mid, lo), jnp.where(ge, hi, mid)

    lo0 = jnp.zeros((r, 1), jnp.int32)
    hi0 = jnp.full((r, 1), jnp.int32(0x7FFFFFFF))
    t, _ = jax.lax.fori_loop(0, SEARCH_BITS, body, (lo0, hi0), unroll=True)

    gt = b > t
    eq = b == t
    n_gt = jnp.sum(gt.astype(jnp.int32), axis=1, keepdims=True)
    n_eq_keep = TOPK - n_gt
    idx = jax.lax.broadcasted_iota(jnp.int32, b.shape, 1)

    def body_idx(_, lohi):
        lo, hi = lohi
        mid = lo + ((hi - lo) >> 1)
        cnt = jnp.sum((eq & (idx < mid)).astype(jnp.int32), axis=1,
                      keepdims=True)
        ge = cnt >= n_eq_keep
        return jnp.where(ge, lo, mid), jnp.where(ge, mid, hi)

    lo0 = jnp.zeros_like(n_gt)
    hi0 = jnp.full_like(n_gt, b.shape[1])
    _, cut = jax.lax.fori_loop(0, 15, body_idx, (lo0, hi0), unroll=True)
    keep = gt | (eq & (idx < cut))
    o_ref[...] = jnp.where(keep, x, jnp.float32(0.0))


@jax.jit
def kernel(x):
    m, n = x.shape
    return pl.pallas_call(
        _abs_topk_block,
        grid=(m // ROWS_PER_BLOCK,),
        in_specs=[pl.BlockSpec((ROWS_PER_BLOCK, n), lambda i: (i, 0))],
        out_specs=pl.BlockSpec((ROWS_PER_BLOCK, n), lambda i: (i, 0)),
        out_shape=jax.ShapeDtypeStruct((m, n), x.dtype),
    )(x)


# speedup vs baseline: 10.4357x; 2.9946x over previous
import jax
import jax.numpy as jnp
from jax.experimental import pallas as pl

TOPK = 64
ROWS_PER_BLOCK = 32
SEARCH_BITS = 31


def _abs_topk_block(x_ref, o_ref):
    x = x_ref[...]
    b = jax.lax.bitcast_convert_type(x, jnp.int32) & jnp.int32(0x7FFFFFFF)
    r = x.shape[0]

    def body(_, lohi):
        lo, hi = lohi
        mid = lo + ((hi - lo) >> 1)
        cnt = jnp.sum((b >= mid).astype(jnp.int32), axis=1, keepdims=True)
        ge = cnt >= TOPK
        return jnp.where(ge, mid, lo), jnp.where(ge, hi, mid)

    lo0 = jnp.zeros((r, 1), jnp.int32)
    hi0 = jnp.full((r, 1), jnp.int32(0x7FFFFFFF))
    t, _ = jax.lax.fori_loop(0, SEARCH_BITS, body, (lo0, hi0), unroll=True)

    gt = b > t
    eq = b == t
    n_gt = jnp.sum(gt.astype(jnp.int32), axis=1, keepdims=True)
    n_eq = jnp.sum(eq.astype(jnp.int32), axis=1, keepdims=True)
    n_eq_keep = TOPK - n_gt

    idx = jax.lax.broadcasted_iota(jnp.int32, b.shape, 1)

    def tie_cutoff():
        def body_idx(_, lohi):
            lo, hi = lohi
            mid = lo + ((hi - lo) >> 1)
            cnt = jnp.sum((eq & (idx < mid)).astype(jnp.int32), axis=1,
                          keepdims=True)
            ge = cnt >= n_eq_keep
            return jnp.where(ge, lo, mid), jnp.where(ge, mid, hi)

        lo0 = jnp.zeros_like(n_gt)
        hi0 = jnp.full_like(n_gt, b.shape[1])
        _, cut = jax.lax.fori_loop(0, 15, body_idx, (lo0, hi0), unroll=True)
        return cut

    cut = jax.lax.cond(jnp.any(n_eq > n_eq_keep), tie_cutoff,
                       lambda: jnp.full_like(n_gt, b.shape[1]))
    keep = gt | (eq & (idx < cut))
    o_ref[...] = jnp.where(keep, x, jnp.float32(0.0))


@jax.jit
def kernel(x):
    m, n = x.shape
    return pl.pallas_call(
        _abs_topk_block,
        grid=(m // ROWS_PER_BLOCK,),
        in_specs=[pl.BlockSpec((ROWS_PER_BLOCK, n), lambda i: (i, 0))],
        out_specs=pl.BlockSpec((ROWS_PER_BLOCK, n), lambda i: (i, 0)),
        out_shape=jax.ShapeDtypeStruct((m, n), x.dtype),
    )(x)
